# Initial kernel scaffold; baseline (speedup 1.0000x reference)
import jax
import jax.numpy as jnp
from jax.experimental import pallas as pl


def kernel(x_student, x_course, x_concept, ew_enroll, ew_engage, ew_rev_enroll, ew_rev_engage, ei_enroll, ei_engage, ei_prereq, ei_covers, ei_cprereq, ei_rev_enroll, ei_rev_engage, params):
    raise NotImplementedError("write your pallas kernel here")


# speedup vs baseline: 1.2939x; 1.2939x over previous
import functools

import jax
import jax.numpy as jnp
from jax.experimental import pallas as pl

NS, NC, NK = 50000, 5000, 2000
H, DC = 64, 128


def _relu_kernel(a_ref, o_ref):
    o_ref[...] = jnp.maximum(a_ref[...], 0.0)


def _relu(x):
    return pl.pallas_call(
        _relu_kernel,
        out_shape=jax.ShapeDtypeStruct(x.shape, x.dtype),
    )(x)


def _gat(x_src, x_dst, ei, ew, Ws, Wd, ats, atd, We, ate, b, n_dst):
    hs = x_src @ Ws
    alpha_s = hs @ ats
    alpha_d = (x_dst @ Wd) @ atd
    c = (We[0] * ate).sum()
    s, d = ei[0], ei[1]
    a = alpha_s[s] + alpha_d[d] + c * ew
    a = jnp.where(a >= 0, a, 0.2 * a)
    ex = jnp.exp(a)
    den = jax.ops.segment_sum(ex, d, num_segments=n_dst)
    coef = ex / (den[d] + 1e-16)
    return jax.ops.segment_sum(hs[s] * coef[:, None], d, num_segments=n_dst) + b


def _sage(x_src, x_dst, ei, Wl, bl, Wr, n_dst):
    s, d = ei[0], ei[1]
    hs = x_src @ Wl
    tot = jax.ops.segment_sum(hs[s], d, num_segments=n_dst)
    cnt = jax.ops.segment_sum(jnp.ones((s.shape[0],), x_src.dtype), d, num_segments=n_dst)
    mean = tot / jnp.clip(cnt, 1.0)[:, None]
    return mean + bl + x_dst @ Wr


def _ln(x, g, b):
    mu = x.mean(-1, keepdims=True)
    var = ((x - mu) ** 2).mean(-1, keepdims=True)
    return (x - mu) / jnp.sqrt(var + 1e-5) * g + b


def kernel(x_student, x_course, x_concept, ew_enroll, ew_engage, ew_rev_enroll, ew_rev_engage, ei_enroll, ei_engage, ei_prereq, ei_covers, ei_cprereq, ei_rev_enroll, ei_rev_engage, params):
    p = params
    cfs = _gat(x_student, x_course, ei_enroll, ew_enroll, p['Ws_en'], p['Wd_en'], p['ats_en'], p['atd_en'], p['We_en'], p['ate_en'], p['b_en'], NC)
    cfc = _sage(x_course, x_course, ei_prereq, p['Wl_pr'], p['bl_pr'], p['Wr_pr'], NC)
    kfs = _gat(x_student, x_concept, ei_engage, ew_engage, p['Ws_eg'], p['Wd_eg'], p['ats_eg'], p['atd_eg'], p['We_eg'], p['ate_eg'], p['b_eg'], NK)
    kfc = _sage(x_course, x_concept, ei_covers, p['Wl_cv'], p['bl_cv'], p['Wr_cv'], NK)
    kfk = _sage(x_concept, x_concept, ei_cprereq, p['Wl_cp'], p['bl_cp'], p['Wr_cp'], NK)
    sfc1 = _gat(x_course, x_student, ei_rev_enroll, ew_rev_enroll, p['Ws_ren'], p['Wd_ren'], p['ats_ren'], p['atd_ren'], p['We_ren'], p['ate_ren'], p['b_ren'], NS)
    sfk1 = _gat(x_concept, x_student, ei_rev_engage, ew_rev_engage, p['Ws_reg'], p['Wd_reg'], p['ats_reg'], p['atd_reg'], p['We_reg'], p['ate_reg'], p['b_reg'], NS)
    skip = (x_student @ p['Wp'] + p['bp']) * 0.3
    x_course_1 = _relu(cfs + cfc)
    x_concept_1 = _relu(kfs + kfc + kfk)
    x_student_1 = _relu(_ln(sfc1 + sfk1 + skip, p['g_ln'], p['b_ln']))
    sfc = _sage(x_course_1, x_student_1, ei_rev_enroll, p['Wl_r1'], p['bl_r1'], p['Wr_r1'], NS)
    sfk = _sage(x_concept_1, x_student_1, ei_rev_engage, p['Wl_r2'], p['bl_r2'], p['Wr_r2'], NS)
    x_student_1 = _relu(sfc + sfk)
    return (x_student_1, x_course_1, x_concept_1)


# speedup vs baseline: 27.6187x; 21.3448x over previous
import functools

import jax
import jax.numpy as jnp
from jax import lax
from jax.experimental import pallas as pl
from jax.experimental.pallas import tpu as pltpu
from jax.experimental.pallas import tpu_sc as plsc

NS, NC, NK = 50000, 5000, 2000
H, DC = 64, 128
L = 16
NT = 16
NSC = 2
EPS = 1e-16
_INTERPRET = False


def _ceil_to(x, m):
    return (x + m - 1) // m * m


def _mm_kernel(x_ref, w_ref, o_ref):
    o_ref[...] = jnp.dot(x_ref[...], w_ref[...], preferred_element_type=jnp.float32)


def _mm(x, w, blk=2048):
    n, k = x.shape
    npad = _ceil_to(n, blk)
    if npad != n:
        x = jnp.pad(x, ((0, npad - n), (0, 0)))
    out = pl.pallas_call(
        _mm_kernel,
        grid=(npad // blk,),
        in_specs=[
            pl.BlockSpec((blk, k), lambda i: (i, 0)),
            pl.BlockSpec((k, w.shape[1]), lambda i: (0, 0)),
        ],
        out_specs=pl.BlockSpec((blk, w.shape[1]), lambda i: (i, 0)),
        out_shape=jax.ShapeDtypeStruct((npad, w.shape[1]), jnp.float32),
        interpret=_INTERPRET,
    )(x, w)
    return out


def _proj_alpha_kernel(x_ref, w_ref, av_ref, h_ref, a_ref):
    h = jnp.dot(x_ref[...], w_ref[...], preferred_element_type=jnp.float32)
    h_ref[...] = h
    a_ref[...] = jnp.dot(h, av_ref[...], preferred_element_type=jnp.float32)


def _proj_alpha(x, w, avec, blk=2048):
    n, k = x.shape
    npad = _ceil_to(n, blk)
    if npad != n:
        x = jnp.pad(x, ((0, npad - n), (0, 0)))
    av = jnp.zeros((H, 128), jnp.float32).at[:, 0].set(avec)
    h, a = pl.pallas_call(
        _proj_alpha_kernel,
        grid=(npad // blk,),
        in_specs=[
            pl.BlockSpec((blk, k), lambda i: (i, 0)),
            pl.BlockSpec((k, H), lambda i: (0, 0)),
            pl.BlockSpec((H, 128), lambda i: (0, 0)),
        ],
        out_specs=[
            pl.BlockSpec((blk, H), lambda i: (i, 0)),
            pl.BlockSpec((blk, 128), lambda i: (i, 0)),
        ],
        out_shape=[
            jax.ShapeDtypeStruct((npad, H), jnp.float32),
            jax.ShapeDtypeStruct((npad, 128), jnp.float32),
        ],
        interpret=_INTERPRET,
    )(x, w, av)
    return h, a[:, 0]


@functools.cache
def _build_sc_pass(n_src, n_dst_a, e_rows_w, n_chunks, chunk, gat,
                   feat_split, rows_on, den_on, cnt_on, width):
    crows = chunk // 128
    mesh = plsc.VectorSubcoreMesh(core_axis_name="c", subcore_axis_name="s")

    in_names = (["hs"] if rows_on else []) + ["si", "di"]
    if gat:
        in_names += ["as_", "ad", "ew"]
    out_names = ((["rowsO"] if rows_on else [])
                 + (["denO"] if den_on else [])
                 + (["cntO"] if cnt_on else []))
    out_type = []
    if rows_on:
        out_type.append(jax.ShapeDtypeStruct((NSC, n_dst_a, width), jnp.float32))
    if den_on:
        out_type.append(jax.ShapeDtypeStruct((NSC, n_dst_a, L), jnp.float32))
    if cnt_on:
        out_type.append(jax.ShapeDtypeStruct((NSC, n_dst_a, L), jnp.float32))

    scr_names, scratch = [], []

    def scr(name, s):
        scr_names.append(name)
        scratch.append(s)

    if rows_on:
        scr("acc", pltpu.VMEM_SHARED((n_dst_a, width), jnp.float32))
    if den_on:
        scr("dacc", pltpu.VMEM_SHARED((n_dst_a, L), jnp.float32))
    if cnt_on:
        scr("cacc", pltpu.VMEM_SHARED((n_dst_a, L), jnp.float32))
    scr("si_v", pltpu.VMEM((crows, 128), jnp.int32))
    scr("di_v", pltpu.VMEM((crows, 128), jnp.int32))
    if rows_on:
        scr("rows_v", pltpu.VMEM((chunk, width), jnp.float32))
    if den_on or cnt_on:
        scr("zr16", pltpu.VMEM((128, L), jnp.float32))
    if cnt_on:
        scr("ones16", pltpu.VMEM((128, L), jnp.float32))
    if gat:
        scr("ew_v", pltpu.VMEM((crows, 128), jnp.float32))
        scr("ex_v", pltpu.VMEM((chunk,), jnp.float32))
        if den_on:
            scr("ex16", pltpu.VMEM((chunk, L), jnp.float32))
        scr("asg_v", pltpu.VMEM((chunk,), jnp.float32))
        scr("adg_v", pltpu.VMEM((chunk,), jnp.float32))
        scr("asem", pltpu.SemaphoreType.DMA)
    scr("gsem", pltpu.SemaphoreType.DMA)
    scr("ssem", pltpu.SemaphoreType.DMA)

    names = in_names + out_names + scr_names

    def body(*refs):
        r = dict(zip(names, refs))
        cid = lax.axis_index("c")
        tid = lax.axis_index("s")
        w_row0 = (tid if feat_split else cid * NT + tid) * e_rows_w
        si_v, di_v = r["si_v"], r["di_v"]
        zvec = jnp.zeros((L,), jnp.float32)

        if rows_on:
            for rr in range(128):
                for k2 in range(width // L):
                    r["rows_v"][rr, pl.ds(k2 * L, L)] = zvec
        if den_on or cnt_on:
            for rr in range(128):
                r["zr16"][rr, pl.ds(0, L)] = zvec
        if cnt_on:
            for rr in range(128):
                r["ones16"][rr, pl.ds(0, L)] = jnp.full((L,), 1.0, jnp.float32)
        rpt = n_dst_a // NT
        for (on, accn, zsrc) in ((rows_on, "acc", "rows_v"),
                                 (den_on, "dacc", "zr16"),
                                 (cnt_on, "cacc", "zr16")):
            if not on:
                continue
            left = rpt
            while left > 0:
                nb = min(128, left)
                pltpu.sync_copy(
                    r[zsrc].at[pl.ds(0, nb)],
                    r[accn].at[pl.ds(tid * rpt + rpt - left, nb)],
                )
                left -= nb
        plsc.subcore_barrier()

        def chunk_fn(i, carry):
            row0 = w_row0 + i * crows
            pltpu.sync_copy(r["si"].at[pl.ds(row0, crows)], si_v)
            pltpu.sync_copy(r["di"].at[pl.ds(row0, crows)], di_v)
            if gat:
                pltpu.sync_copy(r["ew"].at[pl.ds(row0, crows)], r["ew_v"])
            gathers = []
            if rows_on:
                for j in range(crows):
                    src = (r["hs"].at[cid] if feat_split else r["hs"])
                    gathers.append(pltpu.async_copy(
                        src.at[si_v.at[j]],
                        r["rows_v"].at[pl.ds(j * 128, 128)], r["gsem"]))
            if gat:
                agathers = []
                for j in range(crows):
                    agathers.append(pltpu.async_copy(
                        r["as_"].at[si_v.at[j]],
                        r["asg_v"].at[pl.ds(j * 128, 128)], r["asem"]))
                    agathers.append(pltpu.async_copy(
                        r["ad"].at[di_v.at[j]],
                        r["adg_v"].at[pl.ds(j * 128, 128)], r["asem"]))
                for g in agathers:
                    g.wait()
                for j in range(crows):
                    for k2 in range(128 // L):
                        sl = pl.ds(k2 * L, L)
                        g16 = (j * (128 // L) + k2) * L
                        a = (r["asg_v"][pl.ds(g16, L)]
                             + r["adg_v"][pl.ds(g16, L)]
                             + r["ew_v"].at[j][sl])
                        a = jnp.where(a >= 0.0, a, 0.2 * a)
                        r["ex_v"][pl.ds(g16, L)] = jnp.exp(a)
            for g in gathers:
                g.wait()
            if gat and (rows_on or den_on):
                for g in range(chunk // L):
                    exvec = r["ex_v"][pl.ds(g * L, L)]
                    for rr in range(L):
                        rowi = g * L + rr
                        exs = exvec[rr]
                        if den_on:
                            r["ex16"][rowi, pl.ds(0, L)] = \
                                jnp.broadcast_to(exs, (L,))
                        if rows_on:
                            for k2 in range(width // L):
                                sl = pl.ds(k2 * L, L)
                                r["rows_v"][rowi, sl] = \
                                    r["rows_v"][rowi, sl] * exs
            scatters = []
            for j in range(crows):
                idx = di_v.at[j]
                if rows_on:
                    scatters.append(pltpu.async_copy(
                        r["rows_v"].at[pl.ds(j * 128, 128)],
                        r["acc"].at[idx], r["ssem"], add=True))
                if den_on:
                    scatters.append(pltpu.async_copy(
                        r["ex16"].at[pl.ds(j * 128, 128)],
                        r["dacc"].at[idx], r["ssem"], add=True))
                if cnt_on:
                    scatters.append(pltpu.async_copy(
                        r["ones16"], r["cacc"].at[idx], r["ssem"], add=True))
            for s in scatters:
                s.wait()
            return carry

        lax.fori_loop(0, n_chunks, chunk_fn, 0)
        plsc.subcore_barrier()

        for (on, accn, outn) in ((rows_on, "acc", "rowsO"),
                                 (den_on, "dacc", "denO"),
                                 (cnt_on, "cacc", "cntO")):
            if not on:
                continue
            left = rpt
            while left > 0:
                nb = min(128, left)
                off = tid * rpt + rpt - left
                pltpu.sync_copy(
                    r[accn].at[pl.ds(off, nb)],
                    r[outn].at[cid].at[pl.ds(off, nb)],
                )
                left -= nb

    return pl.kernel(
        body, out_type=out_type, mesh=mesh, scratch_types=scratch,
        compiler_params=pltpu.CompilerParams(use_tc_tiling_on_sc=False),
        name=("sc_edge_" + ("gat" if gat else "sage")
              + ("_rows" if rows_on else "")
              + ("_den" if den_on else "") + ("_cnt" if cnt_on else "")
              + f"_{n_dst_a}_{n_src}"),
    )


_EMULATE = set()


def _jnp_rows(hs, ex, si, di, n_src, n_dst, feat_split, width):
    n_dst_a = _ceil_to(n_dst + 1, 2048)
    s = si.reshape(-1)
    d = di.reshape(-1)
    vals = hs[:n_src][s] * (ex[:, None] if ex is not None else 1.0)
    out = jax.ops.segment_sum(vals, d, num_segments=n_dst_a)
    if feat_split:
        rows = jnp.stack([out[:, :32], out[:, 32:]])
    else:
        rows = jnp.stack([out, jnp.zeros_like(out)])
    return rows


def _jnp_ex(a_s, a_d, si, di, ewp, n_src, n_dst):
    s = si.reshape(-1)
    d = di.reshape(-1)
    n_dst_a = _ceil_to(n_dst + 1, 2048)
    ad_pad = jnp.pad(a_d[:n_dst], (0, n_dst_a - n_dst))
    a = a_s[:n_src][s] + ad_pad[d] + ewp.reshape(-1)
    a = jnp.where(a >= 0, a, 0.2 * a)
    return jnp.exp(a)


def _jnp_seg(vals, di, n_dst):
    n_dst_a = _ceil_to(n_dst + 1, 2048)
    den = jax.ops.segment_sum(vals, di.reshape(-1), num_segments=n_dst_a)
    return jnp.stack([den, jnp.zeros_like(den)])


def _prep_edges(ei, n_src, n_dst, ew=None):
    n_dst_a = _ceil_to(n_dst + 1, 2048)
    nw = NSC * NT
    e = ei.shape[1]
    e_pad = nw * _ceil_to(_ceil_to(e, nw) // nw, 512)
    ar = jnp.arange(e_pad - e, dtype=jnp.int32)
    s = jnp.concatenate([ei[0].astype(jnp.int32), ar % n_src]).reshape(-1, 128)
    d = jnp.concatenate(
        [ei[1].astype(jnp.int32), n_dst + ar % (n_dst_a - n_dst)]
    ).reshape(-1, 128)
    if ew is None:
        return s, d, None
    ewp = jnp.concatenate(
        [ew, jnp.zeros((e_pad - e,), jnp.float32)]).reshape(-1, 128)
    return s, d, ewp


def _sc_scalar_gat(a_s, a_d, si, di, ewp, n_src, n_dst):
    n_dst_a = _ceil_to(n_dst + 1, 2048)
    if "scalar" in _EMULATE:
        ex = _jnp_ex(a_s, a_d, si, di, ewp, n_src, n_dst)
        return (_jnp_seg(ex, di, n_dst),
                _jnp_seg(jnp.ones_like(ex), di, n_dst))
    chunk = 512
    e_w = si.shape[0] * 128 // (NSC * NT)
    fn = _build_sc_pass(n_src, n_dst_a, e_w // 128, e_w // chunk, chunk,
                        True, False, False, True, True, 0)
    den, cnt = fn(si, di, a_s[:n_src],
                  jnp.pad(a_d[:n_dst], (0, n_dst_a - n_dst)), ewp)
    return den[:, :, 0], cnt[:, :, 0]


def _sc_gat_rows(hs, a_s, a_d, si, di, ewp, n_src, n_dst, feat_split):
    n_dst_a = _ceil_to(n_dst + 1, 2048)
    if "gat_rows" in _EMULATE:
        ex = _jnp_ex(a_s, a_d, si, di, ewp, n_src, n_dst)
        rows = _jnp_rows(hs, ex, si, di, n_src, n_dst, feat_split,
                         32 if feat_split else 64)
        if feat_split:
            return rows, None
        return rows, _jnp_seg(ex, di, n_dst)
    chunk = 256 if not feat_split else 512
    n_w = NT if feat_split else NSC * NT
    e_w = si.shape[0] * 128 // n_w
    width = 32 if feat_split else 64
    hs_in = (jnp.stack([hs[:n_src, :32], hs[:n_src, 32:]]) if feat_split
             else hs[:n_src])
    fn = _build_sc_pass(n_src, n_dst_a, e_w // 128, e_w // chunk, chunk,
                        True, feat_split, True, not feat_split, False, width)
    outs = fn(hs_in, si, di, a_s[:n_src],
              jnp.pad(a_d[:n_dst], (0, n_dst_a - n_dst)), ewp)
    if feat_split:
        return outs[0], None
    return outs[0], outs[1][:, :, 0]


def _sc_sage_rows(hs, si, di, n_src, n_dst, feat_split):
    n_dst_a = _ceil_to(n_dst + 1, 2048)
    if "sage_rows" in _EMULATE:
        rows = _jnp_rows(hs, None, si, di, n_src, n_dst, feat_split,
                         32 if feat_split else 64)
        if feat_split:
            return rows, None
        ones = jnp.ones((si.size,), jnp.float32)
        return rows, _jnp_seg(ones, di, n_dst)
    chunk = 512
    n_w = NT if feat_split else NSC * NT
    e_w = si.shape[0] * 128 // n_w
    width = 32 if feat_split else 64
    hs_in = (jnp.stack([hs[:n_src, :32], hs[:n_src, 32:]]) if feat_split
             else hs[:n_src])
    fn = _build_sc_pass(n_src, n_dst_a, e_w // 128, e_w // chunk, chunk,
                        False, feat_split, True, False, not feat_split, width)
    outs = fn(hs_in, si, di)
    if feat_split:
        return outs[0], None
    return outs[0], outs[1][:, :, 0]


def _combine_small(terms, x, w, bias, n_dst_a, blk=1024):
    n, k = x.shape
    xp = jnp.pad(x, ((0, n_dst_a - n), (0, 0)))
    flags = tuple(t[2] for t in terms)

    def kern(*refs):
        *term_refs, x_ref, w_ref, b_ref, o_ref = refs
        acc = jnp.dot(x_ref[...], w_ref[...], preferred_element_type=jnp.float32)
        acc = acc + b_ref[...]
        for t, is_gat in enumerate(flags):
            rp = term_refs[2 * t][...]
            den = jnp.sum(term_refs[2 * t + 1][...], axis=0)
            if is_gat:
                acc = acc + (rp[0] + rp[1]) / (den + EPS)[:, None]
            else:
                acc = acc + (rp[0] + rp[1]) / jnp.clip(den, 1.0)[:, None]
        o_ref[...] = jnp.maximum(acc, 0.0)

    in_specs = []
    args = []
    for rows, den, _ in terms:
        in_specs += [
            pl.BlockSpec((2, blk, 64), lambda i: (0, i, 0)),
            pl.BlockSpec((2, blk), lambda i: (0, i)),
        ]
        args += [rows, den]
    in_specs += [
        pl.BlockSpec((blk, k), lambda i: (i, 0)),
        pl.BlockSpec((k, 64), lambda i: (0, 0)),
        pl.BlockSpec((1, 64), lambda i: (0, 0)),
    ]
    args += [xp, w, bias.reshape(1, 64)]
    return pl.pallas_call(
        kern,
        grid=(n_dst_a // blk,),
        in_specs=in_specs,
        out_specs=pl.BlockSpec((blk, 64), lambda i: (i, 0)),
        out_shape=jax.ShapeDtypeStruct((n_dst_a, 64), jnp.float32),
        interpret=_INTERPRET,
    )(*args)


def _combine_student_mid(r_ren, d_ren, r_reg, d_reg, x_stu, wp, bias, g_ln,
                         b_ln, n_dst_a, blk=1024):
    n, k = x_stu.shape
    xp = jnp.pad(x_stu, ((0, n_dst_a - n), (0, 0)))

    def kern(rr_ref, dr_ref, rg_ref, dg_ref, x_ref, w_ref, b_ref, g_ref,
             bl_ref, o_ref):
        skip = 0.3 * jnp.dot(x_ref[...], w_ref[...],
                             preferred_element_type=jnp.float32)
        rr = rr_ref[...]
        cat_ren = jnp.concatenate([rr[0], rr[1]], axis=1)
        den_r = jnp.sum(dr_ref[...], axis=0)
        rg = rg_ref[...]
        cat_reg = jnp.concatenate([rg[0], rg[1]], axis=1)
        den_g = jnp.sum(dg_ref[...], axis=0)
        pre = (cat_ren / (den_r + EPS)[:, None]
               + cat_reg / (den_g + EPS)[:, None] + b_ref[...] + skip)
        mu = jnp.mean(pre, axis=-1, keepdims=True)
        var = jnp.mean((pre - mu) ** 2, axis=-1, keepdims=True)
        y = (pre - mu) / jnp.sqrt(var + 1e-5) * g_ref[...] + bl_ref[...]
        o_ref[...] = jnp.maximum(y, 0.0)

    return pl.pallas_call(
        kern,
        grid=(n_dst_a // blk,),
        in_specs=[
            pl.BlockSpec((2, blk, 32), lambda i: (0, i, 0)),
            pl.BlockSpec((2, blk), lambda i: (0, i)),
            pl.BlockSpec((2, blk, 32), lambda i: (0, i, 0)),
            pl.BlockSpec((2, blk), lambda i: (0, i)),
            pl.BlockSpec((blk, k), lambda i: (i, 0)),
            pl.BlockSpec((k, 64), lambda i: (0, 0)),
            pl.BlockSpec((1, 64), lambda i: (0, 0)),
            pl.BlockSpec((1, 64), lambda i: (0, 0)),
            pl.BlockSpec((1, 64), lambda i: (0, 0)),
        ],
        out_specs=pl.BlockSpec((blk, 64), lambda i: (i, 0)),
        out_shape=jax.ShapeDtypeStruct((n_dst_a, 64), jnp.float32),
        interpret=_INTERPRET,
    )(r_ren, d_ren, r_reg, d_reg, xp, wp, bias.reshape(1, 64),
      g_ln.reshape(1, 64), b_ln.reshape(1, 64))


def _combine_student_out(r_r1, c_r1, r_r2, c_r2, x1, w, bias, n_dst_a,
                         blk=1024):
    def kern(r1_ref, c1_ref, r2_ref, c2_ref, x_ref, w_ref, b_ref, o_ref):
        acc = jnp.dot(x_ref[...], w_ref[...], preferred_element_type=jnp.float32)
        r1 = r1_ref[...]
        cat1 = jnp.concatenate([r1[0], r1[1]], axis=1)
        cnt1 = jnp.clip(jnp.sum(c1_ref[...], axis=0), 1.0)
        r2 = r2_ref[...]
        cat2 = jnp.concatenate([r2[0], r2[1]], axis=1)
        cnt2 = jnp.clip(jnp.sum(c2_ref[...], axis=0), 1.0)
        acc = acc + b_ref[...] + cat1 / cnt1[:, None] + cat2 / cnt2[:, None]
        o_ref[...] = jnp.maximum(acc, 0.0)

    return pl.pallas_call(
        kern,
        grid=(n_dst_a // blk,),
        in_specs=[
            pl.BlockSpec((2, blk, 32), lambda i: (0, i, 0)),
            pl.BlockSpec((2, blk), lambda i: (0, i)),
            pl.BlockSpec((2, blk, 32), lambda i: (0, i, 0)),
            pl.BlockSpec((2, blk), lambda i: (0, i)),
            pl.BlockSpec((blk, 64), lambda i: (i, 0)),
            pl.BlockSpec((64, 64), lambda i: (0, 0)),
            pl.BlockSpec((1, 64), lambda i: (0, 0)),
        ],
        out_specs=pl.BlockSpec((blk, 64), lambda i: (i, 0)),
        out_shape=jax.ShapeDtypeStruct((n_dst_a, 64), jnp.float32),
        interpret=_INTERPRET,
    )(r_r1, c_r1, r_r2, c_r2, x1, w, bias.reshape(1, 64))


def kernel(x_student, x_course, x_concept, ew_enroll, ew_engage,
           ew_rev_enroll, ew_rev_engage, ei_enroll, ei_engage, ei_prereq,
           ei_covers, ei_cprereq, ei_rev_enroll, ei_rev_engage, params):
    p = params
    cs = {k: (p['We_' + k][0] * p['ate_' + k]).sum() for k in
          ('en', 'eg', 'ren', 'reg')}

    hs_en, as_en = _proj_alpha(x_student, p['Ws_en'], p['ats_en'])
    _, ad_en = _proj_alpha(x_course, p['Wd_en'], p['atd_en'])
    hs_eg, as_eg = _proj_alpha(x_student, p['Ws_eg'], p['ats_eg'])
    _, ad_eg = _proj_alpha(x_concept, p['Wd_eg'], p['atd_eg'])
    hs_ren, as_ren = _proj_alpha(x_course, p['Ws_ren'], p['ats_ren'])
    _, ad_ren = _proj_alpha(x_student, p['Wd_ren'], p['atd_ren'])
    hs_reg, as_reg = _proj_alpha(x_concept, p['Ws_reg'], p['ats_reg'])
    _, ad_reg = _proj_alpha(x_student, p['Wd_reg'], p['atd_reg'])
    hp_pr = _mm(x_course, p['Wl_pr'])
    hp_cv = _mm(x_course, p['Wl_cv'])
    hp_cp = _mm(x_concept, p['Wl_cp'])

    si_en, di_en, ew_en = _prep_edges(ei_enroll, NS, NC, cs['en'] * ew_enroll)
    si_eg, di_eg, ew_eg = _prep_edges(ei_engage, NS, NK, cs['eg'] * ew_engage)
    si_ren, di_ren, ew_ren = _prep_edges(
        ei_rev_enroll, NC, NS, cs['ren'] * ew_rev_enroll)
    si_reg, di_reg, ew_reg = _prep_edges(
        ei_rev_engage, NK, NS, cs['reg'] * ew_rev_engage)
    si_pr, di_pr, _ = _prep_edges(ei_prereq, NC, NC)
    si_cv, di_cv, _ = _prep_edges(ei_covers, NC, NK)
    si_cp, di_cp, _ = _prep_edges(ei_cprereq, NK, NK)

    r_en, d_en = _sc_gat_rows(hs_en, as_en, ad_en, si_en, di_en, ew_en,
                              NS, NC, False)
    r_eg, d_eg = _sc_gat_rows(hs_eg, as_eg, ad_eg, si_eg, di_eg, ew_eg,
                              NS, NK, False)
    r_ren, _ = _sc_gat_rows(hs_ren, as_ren, ad_ren, si_ren, di_ren, ew_ren,
                            NC, NS, True)
    r_reg, _ = _sc_gat_rows(hs_reg, as_reg, ad_reg, si_reg, di_reg, ew_reg,
                            NK, NS, True)
    d_ren, cnt_r1 = _sc_scalar_gat(as_ren, ad_ren, si_ren, di_ren, ew_ren,
                                   NC, NS)
    d_reg, cnt_r2 = _sc_scalar_gat(as_reg, ad_reg, si_reg, di_reg, ew_reg,
                                   NK, NS)
    r_pr, c_pr = _sc_sage_rows(hp_pr, si_pr, di_pr, NC, NC, False)
    r_cv, c_cv = _sc_sage_rows(hp_cv, si_cv, di_cv, NC, NK, False)
    r_cp, c_cp = _sc_sage_rows(hp_cp, si_cp, di_cp, NK, NK, False)

    nca = _ceil_to(NC + 1, 2048)
    nka = _ceil_to(NK + 1, 2048)
    nsa = _ceil_to(NS + 1, 2048)
    x_course_1 = _combine_small(
        [(r_en, d_en, True), (r_pr, c_pr, False)],
        x_course, p['Wr_pr'], p['b_en'] + p['bl_pr'], nca)
    x_concept_1 = _combine_small(
        [(r_eg, d_eg, True), (r_cv, c_cv, False), (r_cp, c_cp, False)],
        x_concept, p['Wr_cv'] + p['Wr_cp'], p['b_eg'] + p['bl_cv'] + p['bl_cp'],
        nka)
    x_student_1 = _combine_student_mid(
        r_ren, d_ren, r_reg, d_reg, x_student, p['Wp'],
        p['b_ren'] + p['b_reg'] + 0.3 * p['bp'], p['g_ln'], p['b_ln'], nsa)

    hp_r1 = _mm(x_course_1[:NC], p['Wl_r1'])
    hp_r2 = _mm(x_concept_1[:NK], p['Wl_r2'])
    r_r1, _ = _sc_sage_rows(hp_r1, si_ren, di_ren, NC, NS, True)
    r_r2, _ = _sc_sage_rows(hp_r2, si_reg, di_reg, NK, NS, True)

    x_student_out = _combine_student_out(
        r_r1, cnt_r1, r_r2, cnt_r2, x_student_1, p['Wr_r1'] + p['Wr_r2'],
        p['bl_r1'] + p['bl_r2'], nsa)

    return (x_student_out[:NS], x_course_1[:NC], x_concept_1[:NK])
